# Initial kernel scaffold; baseline (speedup 1.0000x reference)
import jax
import jax.numpy as jnp
from jax.experimental import pallas as pl


def kernel(x, edge_index, Ws1, Wn1, b1, Ws2, Wn2, b2, Ws3, Wn3, b3):
    raise NotImplementedError("write your pallas kernel here")


# speedup vs baseline: 7.6705x; 7.6705x over previous
import jax
import jax.numpy as jnp
from jax import lax
from jax.experimental import pallas as pl
from jax.experimental.pallas import tpu as pltpu
from jax.experimental.pallas import tpu_sc as plsc

N = 10000
E = 320000
D_IN = 128
D_HID = 128
D_OUT = 47
D_OUT_PAD = 48
D_AUG = 144

NC = 2
NS = 16
NW = NC * NS
EPW = E // NW
C = 80
G = EPW // C
G2 = 25
G1 = G // G2
RPT = 624
TAIL = N - NS * RPT

_mesh = plsc.VectorSubcoreMesh(
    core_axis_name="c", subcore_axis_name="s", num_cores=NC, num_subcores=NS)


def _make_sc_agg(d):
  scratch = [
      pltpu.VMEM((G2, C), jnp.int32),
      pltpu.VMEM((G2, C), jnp.int32),
      pltpu.VMEM((C, d), jnp.float32),
      pltpu.SemaphoreType.DMA,
      pltpu.VMEM_SHARED((N, d), jnp.float32),
  ]

  def body(table, src3, dst3, acc_out, src_v, dst_v, rows_v, sem, acc_s):
    c = lax.axis_index("c")
    s = lax.axis_index("s")
    wid = s * NC + c
    base = s * RPT

    def initrow(i, carry):
      for k in range(d // 16):
        rows_v[i, pl.ds(k * 16, 16)] = jnp.zeros((16,), jnp.float32)
      return carry
    lax.fori_loop(0, C, initrow, 0)

    for k in range(7):
      pltpu.sync_copy(rows_v, acc_s.at[pl.ds(base + k * C, C)])
    pltpu.sync_copy(rows_v.at[pl.ds(0, RPT - 7 * C)],
                    acc_s.at[pl.ds(base + 7 * C, RPT - 7 * C)])

    @pl.when(s == NS - 1)
    def _zero_tail():
      pltpu.sync_copy(rows_v.at[pl.ds(0, TAIL)],
                      acc_s.at[pl.ds(NS * RPT, TAIL)])
    plsc.subcore_barrier()

    def outer(o, carry):
      pltpu.sync_copy(src3.at[wid, o], src_v)
      pltpu.sync_copy(dst3.at[wid, o], dst_v)

      def step(j, carry2):
        pltpu.async_copy(table.at[src_v.at[j]], rows_v, sem).wait()
        pltpu.sync_copy(rows_v, acc_s.at[dst_v.at[j]], add=True)
        return carry2
      return lax.fori_loop(0, G2, step, carry)
    lax.fori_loop(0, G1, outer, 0)
    plsc.subcore_barrier()

    pltpu.sync_copy(acc_s.at[pl.ds(base, RPT)], acc_out.at[c, pl.ds(base, RPT)])

    @pl.when(s == NS - 1)
    def _write_tail():
      pltpu.sync_copy(acc_s.at[pl.ds(NS * RPT, TAIL)],
                      acc_out.at[c, pl.ds(NS * RPT, TAIL)])

  params = None
  if d % 128 != 0:
    params = pltpu.CompilerParams(use_tc_tiling_on_sc=False)
  return pl.kernel(
      body, out_type=(jax.ShapeDtypeStruct((NC, N, d), jnp.float32),),
      mesh=_mesh, scratch_types=tuple(scratch), compiler_params=params)


_sc_agg144 = _make_sc_agg(D_AUG)
_sc_agg = _make_sc_agg(D_HID)
_sc_agg48 = _make_sc_agg(D_OUT_PAD)

_RB = 1000


def _tc1_body(x_ref, acc_ref, ws_ref, wn_ref, b_ref, h_ref, rcp_ref):
  deg = acc_ref[0, :, D_HID:D_HID + 1] + acc_ref[1, :, D_HID:D_HID + 1]
  rcp = 1.0 / jnp.maximum(deg, 1.0)
  agg = (acc_ref[0, :, :D_HID] + acc_ref[1, :, :D_HID]) * rcp
  h = (jnp.dot(x_ref[...], ws_ref[...], preferred_element_type=jnp.float32)
       + jnp.dot(agg, wn_ref[...], preferred_element_type=jnp.float32)
       + b_ref[...])
  h_ref[...] = jnp.maximum(h, 0.0)
  rcp_ref[...] = jnp.broadcast_to(rcp, (_RB, 16))


def _tc2_body(h_ref, acc_ref, rcp_ref, ws_ref, wn_ref, b_ref,
              ws3_ref, wn3_ref, b3_ref, p2_ref, s3_ref):
  rcp = rcp_ref[:, 0:1]
  agg = (acc_ref[0] + acc_ref[1]) * rcp
  h2 = (jnp.dot(h_ref[...], ws_ref[...], preferred_element_type=jnp.float32)
        + jnp.dot(agg, wn_ref[...], preferred_element_type=jnp.float32)
        + b_ref[...])
  h2 = jnp.maximum(h2, 0.0)
  p2_ref[...] = jnp.dot(h2, wn3_ref[...], preferred_element_type=jnp.float32)
  s3_ref[...] = (jnp.dot(h2, ws3_ref[...], preferred_element_type=jnp.float32)
                 + b3_ref[...])


def _tc3_body(s3_ref, acc_ref, rcp_ref, o_ref):
  rcp = rcp_ref[:, 0:1]
  res = s3_ref[...] + (acc_ref[0] + acc_ref[1]) * rcp
  o_ref[...] = res[:, :D_OUT]


def _row_spec(d):
  return pl.BlockSpec((_RB, d), lambda i: (i, 0))


def _acc_spec(d):
  return pl.BlockSpec((NC, _RB, d), lambda i: (0, i, 0))


def _full_spec(r, c):
  return pl.BlockSpec((r, c), lambda i: (0, 0))


_GRID = (N // _RB,)

_tc1 = pl.pallas_call(
    _tc1_body,
    grid=_GRID,
    in_specs=[_row_spec(D_IN), _acc_spec(D_AUG),
              _full_spec(D_IN, D_HID), _full_spec(D_IN, D_HID),
              _full_spec(1, D_HID)],
    out_specs=[_row_spec(D_HID), _row_spec(16)],
    out_shape=[jax.ShapeDtypeStruct((N, D_HID), jnp.float32),
               jax.ShapeDtypeStruct((N, 16), jnp.float32)],
)

_tc2 = pl.pallas_call(
    _tc2_body,
    grid=_GRID,
    in_specs=[_row_spec(D_HID), _acc_spec(D_HID), _row_spec(16),
              _full_spec(D_HID, D_HID), _full_spec(D_HID, D_HID),
              _full_spec(1, D_HID),
              _full_spec(D_HID, D_OUT_PAD), _full_spec(D_HID, D_OUT_PAD),
              _full_spec(1, D_OUT_PAD)],
    out_specs=[_row_spec(D_OUT_PAD), _row_spec(D_OUT_PAD)],
    out_shape=[jax.ShapeDtypeStruct((N, D_OUT_PAD), jnp.float32),
               jax.ShapeDtypeStruct((N, D_OUT_PAD), jnp.float32)],
)

_tc3 = pl.pallas_call(
    _tc3_body,
    grid=_GRID,
    in_specs=[_row_spec(D_OUT_PAD), _acc_spec(D_OUT_PAD), _row_spec(16)],
    out_specs=_row_spec(D_OUT),
    out_shape=jax.ShapeDtypeStruct((N, D_OUT), jnp.float32),
)


def kernel(x, edge_index, Ws1, Wn1, b1, Ws2, Wn2, b2, Ws3, Wn3, b3):
  src3 = edge_index[0].reshape(NW, G1, G2, C)
  dst3 = edge_index[1].reshape(NW, G1, G2, C)

  t1 = jnp.concatenate(
      [x, jnp.ones((N, D_AUG - D_HID), jnp.float32)], axis=1)
  (acc1,) = _sc_agg144(t1, src3, dst3)
  h1, rcp = _tc1(x, acc1, Ws1, Wn1, b1.reshape(1, -1))

  (acc2,) = _sc_agg(h1, src3, dst3)
  ws3p = jnp.pad(Ws3, ((0, 0), (0, D_OUT_PAD - D_OUT)))
  wn3p = jnp.pad(Wn3, ((0, 0), (0, D_OUT_PAD - D_OUT)))
  b3p = jnp.pad(b3, (0, D_OUT_PAD - D_OUT)).reshape(1, -1)
  p2, s3 = _tc2(h1, acc2, rcp, Ws2, Wn2, b2.reshape(1, -1), ws3p, wn3p, b3p)

  (acc3,) = _sc_agg48(p2, src3, dst3)
  out = _tc3(s3, acc3, rcp)
  return out


# speedup vs baseline: 9.3415x; 1.2178x over previous
import jax
import jax.numpy as jnp
from jax import lax
from jax.experimental import pallas as pl
from jax.experimental.pallas import tpu as pltpu
from jax.experimental.pallas import tpu_sc as plsc

N = 10000
E = 320000
D_IN = 128
D_HID = 128
D_OUT = 47
D_OUT_PAD = 48
D_AUG = 144

NC = 2
NS = 16
NW = NC * NS
EPW = E // NW
C = 80
G = EPW // C
G2 = 25
G1 = G // G2
RPT = 624
TAIL = N - NS * RPT

_mesh = plsc.VectorSubcoreMesh(
    core_axis_name="c", subcore_axis_name="s", num_cores=NC, num_subcores=NS)


def _make_sc_agg(d):
  scratch = [
      pltpu.VMEM((G2, C), jnp.int32),
      pltpu.VMEM((G2, C), jnp.int32),
      pltpu.VMEM((2, C, d), jnp.float32),
      pltpu.SemaphoreType.DMA,
      pltpu.SemaphoreType.DMA,
      pltpu.VMEM_SHARED((N, d), jnp.float32),
  ]

  def body(table, src3, dst3, acc_out, src_v, dst_v, rows_v, sem_g, sem_s,
           acc_s):
    c = lax.axis_index("c")
    s = lax.axis_index("s")
    wid = s * NC + c
    base = s * RPT

    def initrow(i, carry):
      for k in range(d // 16):
        rows_v[0, i, pl.ds(k * 16, 16)] = jnp.zeros((16,), jnp.float32)
      return carry
    lax.fori_loop(0, C, initrow, 0)

    zsrc = rows_v.at[0]
    for k in range(7):
      pltpu.sync_copy(zsrc, acc_s.at[pl.ds(base + k * C, C)])
    pltpu.sync_copy(zsrc.at[pl.ds(0, RPT - 7 * C)],
                    acc_s.at[pl.ds(base + 7 * C, RPT - 7 * C)])

    @pl.when(s == NS - 1)
    def _zero_tail():
      pltpu.sync_copy(zsrc.at[pl.ds(0, TAIL)],
                      acc_s.at[pl.ds(NS * RPT, TAIL)])
    plsc.subcore_barrier()

    def outer(o, carry):
      pltpu.sync_copy(src3.at[wid, o], src_v)
      pltpu.sync_copy(dst3.at[wid, o], dst_v)
      pltpu.async_copy(table.at[src_v.at[0]], rows_v.at[0], sem_g)

      def step(j, carry2):
        p = lax.rem(j, 2)
        q = 1 - p
        pltpu.make_async_copy(table.at[src_v.at[j]], rows_v.at[p],
                              sem_g).wait()

        @pl.when(j > 0)
        def _drain_prev_scatter():
          pltpu.make_async_copy(rows_v.at[q], acc_s.at[dst_v.at[j]],
                                sem_s).wait()

        @pl.when(j < G2 - 1)
        def _issue_next_gather():
          pltpu.async_copy(table.at[src_v.at[j + 1]], rows_v.at[q], sem_g)

        pltpu.async_copy(rows_v.at[p], acc_s.at[dst_v.at[j]], sem_s,
                         add=True)
        return carry2
      lax.fori_loop(0, G2, step, carry)
      pltpu.make_async_copy(rows_v.at[0], acc_s.at[dst_v.at[0]], sem_s).wait()
      return carry
    lax.fori_loop(0, G1, outer, 0)
    plsc.subcore_barrier()

    pltpu.sync_copy(acc_s.at[pl.ds(base, RPT)], acc_out.at[c, pl.ds(base, RPT)])

    @pl.when(s == NS - 1)
    def _write_tail():
      pltpu.sync_copy(acc_s.at[pl.ds(NS * RPT, TAIL)],
                      acc_out.at[c, pl.ds(NS * RPT, TAIL)])

  params = None
  if d % 128 != 0:
    params = pltpu.CompilerParams(use_tc_tiling_on_sc=False)
  return pl.kernel(
      body, out_type=(jax.ShapeDtypeStruct((NC, N, d), jnp.float32),),
      mesh=_mesh, scratch_types=tuple(scratch), compiler_params=params)


_sc_agg144 = _make_sc_agg(D_AUG)
_sc_agg = _make_sc_agg(D_HID)
_sc_agg48 = _make_sc_agg(D_OUT_PAD)

_RB = 1000


def _tc1_body(x_ref, acc_ref, ws_ref, wn_ref, b_ref, h_ref, rcp_ref):
  deg = acc_ref[0, :, D_HID:D_HID + 1] + acc_ref[1, :, D_HID:D_HID + 1]
  rcp = 1.0 / jnp.maximum(deg, 1.0)
  agg = (acc_ref[0, :, :D_HID] + acc_ref[1, :, :D_HID]) * rcp
  h = (jnp.dot(x_ref[...], ws_ref[...], preferred_element_type=jnp.float32)
       + jnp.dot(agg, wn_ref[...], preferred_element_type=jnp.float32)
       + b_ref[...])
  h_ref[...] = jnp.maximum(h, 0.0)
  rcp_ref[...] = jnp.broadcast_to(rcp, (_RB, 16))


def _tc2_body(h_ref, acc_ref, rcp_ref, ws_ref, wn_ref, b_ref,
              ws3_ref, wn3_ref, b3_ref, p2_ref, s3_ref):
  rcp = rcp_ref[:, 0:1]
  agg = (acc_ref[0] + acc_ref[1]) * rcp
  h2 = (jnp.dot(h_ref[...], ws_ref[...], preferred_element_type=jnp.float32)
        + jnp.dot(agg, wn_ref[...], preferred_element_type=jnp.float32)
        + b_ref[...])
  h2 = jnp.maximum(h2, 0.0)
  p2_ref[...] = jnp.dot(h2, wn3_ref[...], preferred_element_type=jnp.float32)
  s3_ref[...] = (jnp.dot(h2, ws3_ref[...], preferred_element_type=jnp.float32)
                 + b3_ref[...])


def _tc3_body(s3_ref, acc_ref, rcp_ref, o_ref):
  rcp = rcp_ref[:, 0:1]
  res = s3_ref[...] + (acc_ref[0] + acc_ref[1]) * rcp
  o_ref[...] = res[:, :D_OUT]


def _row_spec(d):
  return pl.BlockSpec((_RB, d), lambda i: (i, 0))


def _acc_spec(d):
  return pl.BlockSpec((NC, _RB, d), lambda i: (0, i, 0))


def _full_spec(r, c):
  return pl.BlockSpec((r, c), lambda i: (0, 0))


_GRID = (N // _RB,)

_tc1 = pl.pallas_call(
    _tc1_body,
    grid=_GRID,
    in_specs=[_row_spec(D_IN), _acc_spec(D_AUG),
              _full_spec(D_IN, D_HID), _full_spec(D_IN, D_HID),
              _full_spec(1, D_HID)],
    out_specs=[_row_spec(D_HID), _row_spec(16)],
    out_shape=[jax.ShapeDtypeStruct((N, D_HID), jnp.float32),
               jax.ShapeDtypeStruct((N, 16), jnp.float32)],
)

_tc2 = pl.pallas_call(
    _tc2_body,
    grid=_GRID,
    in_specs=[_row_spec(D_HID), _acc_spec(D_HID), _row_spec(16),
              _full_spec(D_HID, D_HID), _full_spec(D_HID, D_HID),
              _full_spec(1, D_HID),
              _full_spec(D_HID, D_OUT_PAD), _full_spec(D_HID, D_OUT_PAD),
              _full_spec(1, D_OUT_PAD)],
    out_specs=[_row_spec(D_OUT_PAD), _row_spec(D_OUT_PAD)],
    out_shape=[jax.ShapeDtypeStruct((N, D_OUT_PAD), jnp.float32),
               jax.ShapeDtypeStruct((N, D_OUT_PAD), jnp.float32)],
)

_tc3 = pl.pallas_call(
    _tc3_body,
    grid=_GRID,
    in_specs=[_row_spec(D_OUT_PAD), _acc_spec(D_OUT_PAD), _row_spec(16)],
    out_specs=_row_spec(D_OUT),
    out_shape=jax.ShapeDtypeStruct((N, D_OUT), jnp.float32),
)


def kernel(x, edge_index, Ws1, Wn1, b1, Ws2, Wn2, b2, Ws3, Wn3, b3):
  src3 = edge_index[0].reshape(NW, G1, G2, C)
  dst3 = edge_index[1].reshape(NW, G1, G2, C)

  t1 = jnp.concatenate(
      [x, jnp.ones((N, D_AUG - D_HID), jnp.float32)], axis=1)
  (acc1,) = _sc_agg144(t1, src3, dst3)
  h1, rcp = _tc1(x, acc1, Ws1, Wn1, b1.reshape(1, -1))

  (acc2,) = _sc_agg(h1, src3, dst3)
  ws3p = jnp.pad(Ws3, ((0, 0), (0, D_OUT_PAD - D_OUT)))
  wn3p = jnp.pad(Wn3, ((0, 0), (0, D_OUT_PAD - D_OUT)))
  b3p = jnp.pad(b3, (0, D_OUT_PAD - D_OUT)).reshape(1, -1)
  p2, s3 = _tc2(h1, acc2, rcp, Ws2, Wn2, b2.reshape(1, -1), ws3p, wn3p, b3p)

  (acc3,) = _sc_agg48(p2, src3, dst3)
  out = _tc3(s3, acc3, rcp)
  return out
